# baseline (speedup 1.0000x reference)
import functools

import jax
import jax.numpy as jnp
from jax import lax
from jax.experimental import pallas as pl
from jax.experimental.pallas import tpu as pltpu
from jax.experimental.pallas import tpu_sc as plsc

P = 8192
R = 32
STEPS = P // R
NW = 32
BPW = P // NW

_KS0 = 0
_KS1 = 2
_KS2 = 0x1BD11BDA ^ _KS0 ^ _KS1
_ROT = ((13, 15, 26, 6), (17, 29, 16, 24))
_INJ = ((_KS1, _KS2 + 1), (_KS2, _KS0 + 2), (_KS0, _KS1 + 3),
        (_KS1, _KS2 + 4), (_KS2, _KS0 + 5))
_TINY = 1.1754944e-38


def _rotl(x, r):
    return (x << jnp.uint32(r)) | (x >> jnp.uint32(32 - r))


def _threefry_bits_pre(x1):
    x0 = x1
    x1 = _rotl(x1, _ROT[0][0]) ^ x0
    for r in _ROT[0][1:]:
        x0 = x0 + x1
        x1 = _rotl(x1, r) ^ x0
    x0 = x0 + jnp.uint32(_INJ[0][0])
    x1 = x1 + jnp.uint32(_INJ[0][1])
    for g in range(1, 5):
        for r in _ROT[g % 2]:
            x0 = x0 + x1
            x1 = _rotl(x1, r) ^ x0
        x0 = x0 + jnp.uint32(_INJ[g][0])
        x1 = x1 + jnp.uint32(_INJ[g][1])
    return x0 ^ x1


def _prologue_body(x_ref, svt_ref, nzt_ref, tm_ref, cov_ref, fwd_ref,
                   updt_ref, rinv_ref):
    l00 = jnp.sqrt(cov_ref[0, 0])
    l10 = cov_ref[1, 0] / l00
    l20 = cov_ref[2, 0] / l00
    l11 = jnp.sqrt(cov_ref[1, 1] - l10 * l10)
    l21 = (cov_ref[2, 1] - l20 * l10) / l11
    l22 = jnp.sqrt(cov_ref[2, 2] - l20 * l20 - l21 * l21)

    sv0 = svt_ref[0:1, :]
    sv1 = svt_ref[1:2, :]
    sv2 = svt_ref[2:3, :]
    nz0 = nzt_ref[0:1, :]
    nz1 = nzt_ref[1:2, :]
    nz2 = nzt_ref[2:3, :]

    def trans(k):
        return tm_ref[k, 0] * sv0 + tm_ref[k, 1] * sv1 + tm_ref[k, 2] * sv2

    upd0 = trans(0) + nz0 * l00 + nz1 * l10 + nz2 * l20
    upd1 = trans(1) + nz1 * l11 + nz2 * l21
    upd2 = trans(2) + nz2 * l22
    updt_ref[0:1, :] = upd0
    updt_ref[1:2, :] = upd1
    updt_ref[2:3, :] = upd2

    p = fwd_ref[0, 0] * upd0 + fwd_ref[0, 1] * upd1 + fwd_ref[0, 2] * upd2
    xv = x_ref[...]
    s1 = jnp.sum(xv)
    s2 = jnp.sum(xv * xv)
    n = jnp.float32(xv.shape[1])
    w = s2 - (2.0 * s1) * p + n * (p * p)
    rinv_ref[...] = 0.6931471805599453 / w


def _argmax_first(t, idx_ref):
    m = jnp.max(t, axis=1, keepdims=True)
    cols = lax.broadcasted_iota(jnp.int32, t.shape, 1)
    idx_ref[...] = jnp.min(jnp.where(t == m, cols, jnp.int32(2 ** 30)),
                           axis=1, keepdims=True)


def _bits_to_t(bits, rinv2):
    fb = lax.bitcast_convert_type(
        (bits >> jnp.uint32(9)) | jnp.uint32(0x3F800000), jnp.float32)
    return jnp.log2(fb - 1.0) * rinv2


def _sample_body(rinv2_ref, idx_ref):
    i0 = pl.program_id(0) * R
    base = (i0 * P + _KS1).astype(jnp.uint32)
    n = (lax.broadcasted_iota(jnp.uint32, (R, P), 0) * jnp.uint32(P)
         + lax.broadcasted_iota(jnp.uint32, (R, P), 1)) + base
    t = _bits_to_t(_threefry_bits_pre(n), rinv2_ref[...])
    _argmax_first(t, idx_ref)


def _sample_scbits_body(rinv2_ref, bits_ref, idx_ref):
    t = _bits_to_t(bits_ref[...], rinv2_ref[...])
    _argmax_first(t, idx_ref)


RTC = 4864
RS = P - RTC
RPT = RS // NW
RB2 = 64

_CHUNKS = BPW // 128


@functools.cache
def _make_sc_bits():
    mesh = plsc.VectorSubcoreMesh(core_axis_name="c", subcore_axis_name="s")

    @functools.partial(
        pl.kernel,
        mesh=mesh,
        out_type=jax.ShapeDtypeStruct((RS, P), jnp.uint32),
        scratch_types=[pltpu.VMEM((2, P), jnp.uint32), pltpu.SemaphoreType.DMA],
    )
    def _sc_bits(out_hbm, bufs_v, sem):
        wid = lax.axis_index("s") * 2 + lax.axis_index("c")
        lane = lax.iota(jnp.uint32, 16)

        def row_body(r, carry):
            lrow = wid * RPT + r
            slot = lax.rem(r, 2)
            buf = bufs_v.at[slot]
            @pl.when(r >= 2)
            def _():
                pltpu.make_async_copy(buf, out_hbm.at[lrow], sem).wait()

            nbase = ((RTC + lrow) * P + _KS1).astype(jnp.uint32)

            def grp(g, carry2):
                goff = (g * 256).astype(jnp.uint32)
                for k in range(16):
                    x1 = lane + (nbase + goff + jnp.uint32(k * 16))
                    buf[pl.ds(g * 256 + k * 16, 16)] = _threefry_bits_pre(x1)
                return carry2

            lax.fori_loop(0, P // 256, grp, 0)
            pltpu.async_copy(buf, out_hbm.at[lrow], sem)
            return carry

        lax.fori_loop(0, RPT, row_body, 0)
        pltpu.make_async_copy(bufs_v.at[0], out_hbm.at[wid * RPT], sem).wait()
        pltpu.make_async_copy(bufs_v.at[1], out_hbm.at[wid * RPT], sem).wait()

    return _sc_bits


@functools.cache
def _make_gather_mean():
    mesh = plsc.VectorSubcoreMesh(core_axis_name="c", subcore_axis_name="s")

    @functools.partial(
        pl.kernel,
        mesh=mesh,
        out_type=jax.ShapeDtypeStruct((NW, 16), jnp.float32),
        scratch_types=[
            pltpu.VMEM((_CHUNKS, 128), jnp.int32),
            pltpu.VMEM((_CHUNKS, 128, 128), jnp.float32),
            pltpu.VMEM((16,), jnp.float32),
            pltpu.SemaphoreType.DMA,
        ],
    )
    def _gather_mean(table_hbm, idx_hbm, out_hbm, idx_v, rows_v, acc_v, sem):
        wid = lax.axis_index("s") * 2 + lax.axis_index("c")
        pltpu.sync_copy(idx_hbm.at[pl.ds(wid * _CHUNKS, _CHUNKS)], idx_v)
        for j in range(_CHUNKS):
            pltpu.async_copy(table_hbm.at[idx_v.at[j]], rows_v.at[j], sem).wait()
        acc = jnp.zeros((16,), jnp.float32)
        for j in range(_CHUNKS):
            for g in range(128):
                acc = acc + rows_v[j, g, pl.ds(0, 16)]
        acc_v[...] = acc
        pltpu.sync_copy(acc_v, out_hbm.at[wid])

    return _gather_mean


def kernel(inputs, state_vector, transition_matrix, process_noise_cov, forward_matrix):
    x = inputs.astype(jnp.float32)
    noise = jax.random.normal(jax.random.key(1), state_vector.shape, jnp.float32)
    svt = state_vector.T
    nzt = noise.T

    updt, rinv = pl.pallas_call(
        _prologue_body,
        in_specs=[
            pl.BlockSpec(memory_space=pltpu.VMEM),
            pl.BlockSpec(memory_space=pltpu.VMEM),
            pl.BlockSpec(memory_space=pltpu.VMEM),
            pl.BlockSpec(memory_space=pltpu.SMEM),
            pl.BlockSpec(memory_space=pltpu.SMEM),
            pl.BlockSpec(memory_space=pltpu.SMEM),
        ],
        out_specs=[
            pl.BlockSpec(memory_space=pltpu.VMEM),
            pl.BlockSpec(memory_space=pltpu.VMEM),
        ],
        out_shape=[
            jax.ShapeDtypeStruct((3, P), jnp.float32),
            jax.ShapeDtypeStruct((1, P), jnp.float32),
        ],
    )(x, svt, nzt, transition_matrix, process_noise_cov, forward_matrix)

    scbits = _make_sc_bits()()

    idx1 = pl.pallas_call(
        _sample_body,
        grid=(RTC // R,),
        in_specs=[pl.BlockSpec((1, P), lambda s: (0, 0))],
        out_specs=pl.BlockSpec((R, 1), lambda s: (s, 0)),
        out_shape=jax.ShapeDtypeStruct((RTC, 1), jnp.int32),
    )(rinv)

    idx2 = pl.pallas_call(
        _sample_scbits_body,
        grid=(RS // RB2,),
        in_specs=[pl.BlockSpec((1, P), lambda s: (0, 0)),
                  pl.BlockSpec((RB2, P), lambda s: (s, 0))],
        out_specs=pl.BlockSpec((RB2, 1), lambda s: (s, 0)),
        out_shape=jax.ShapeDtypeStruct((RS, 1), jnp.int32),
    )(rinv, scbits)

    idx2d = jnp.concatenate([idx1, idx2], axis=0)
    table = jnp.zeros((P, 128), jnp.float32).at[:, :3].set(updt.T)
    partials = _make_gather_mean()(table, idx2d.reshape(NW * _CHUNKS, 128))
    return jnp.sum(partials, axis=0)[:3] / jnp.float32(P)


# speedup vs baseline: 1.1267x; 1.1267x over previous
import functools

import jax
import jax.numpy as jnp
from jax import lax
from jax.experimental import pallas as pl
from jax.experimental.pallas import tpu as pltpu
from jax.experimental.pallas import tpu_sc as plsc

P = 8192
R = 32
STEPS = P // R
NW = 32
BPW = P // NW

_KS0 = 0
_KS1 = 2
_KS2 = 0x1BD11BDA ^ _KS0 ^ _KS1
_ROT = ((13, 15, 26, 6), (17, 29, 16, 24))
_INJ = ((_KS1, _KS2 + 1), (_KS2, _KS0 + 2), (_KS0, _KS1 + 3),
        (_KS1, _KS2 + 4), (_KS2, _KS0 + 5))
_TINY = 1.1754944e-38


def _rotl(x, r):
    return (x << jnp.uint32(r)) | (x >> jnp.uint32(32 - r))


def _threefry_bits_pre(x1):
    x0 = x1
    x1 = _rotl(x1, _ROT[0][0]) ^ x0
    for r in _ROT[0][1:]:
        x0 = x0 + x1
        x1 = _rotl(x1, r) ^ x0
    x0 = x0 + jnp.uint32(_INJ[0][0])
    x1 = x1 + jnp.uint32(_INJ[0][1])
    for g in range(1, 5):
        for r in _ROT[g % 2]:
            x0 = x0 + x1
            x1 = _rotl(x1, r) ^ x0
        x0 = x0 + jnp.uint32(_INJ[g][0])
        x1 = x1 + jnp.uint32(_INJ[g][1])
    return x0 ^ x1


def _prologue_body(x_ref, svt_ref, nzt_ref, tm_ref, cov_ref, fwd_ref,
                   updt_ref, rinv_ref):
    l00 = jnp.sqrt(cov_ref[0, 0])
    l10 = cov_ref[1, 0] / l00
    l20 = cov_ref[2, 0] / l00
    l11 = jnp.sqrt(cov_ref[1, 1] - l10 * l10)
    l21 = (cov_ref[2, 1] - l20 * l10) / l11
    l22 = jnp.sqrt(cov_ref[2, 2] - l20 * l20 - l21 * l21)

    sv0 = svt_ref[0:1, :]
    sv1 = svt_ref[1:2, :]
    sv2 = svt_ref[2:3, :]
    nz0 = nzt_ref[0:1, :]
    nz1 = nzt_ref[1:2, :]
    nz2 = nzt_ref[2:3, :]

    def trans(k):
        return tm_ref[k, 0] * sv0 + tm_ref[k, 1] * sv1 + tm_ref[k, 2] * sv2

    upd0 = trans(0) + nz0 * l00 + nz1 * l10 + nz2 * l20
    upd1 = trans(1) + nz1 * l11 + nz2 * l21
    upd2 = trans(2) + nz2 * l22
    updt_ref[0:1, :] = upd0
    updt_ref[1:2, :] = upd1
    updt_ref[2:3, :] = upd2

    p = fwd_ref[0, 0] * upd0 + fwd_ref[0, 1] * upd1 + fwd_ref[0, 2] * upd2
    xv = x_ref[...]
    s1 = jnp.sum(xv)
    s2 = jnp.sum(xv * xv)
    n = jnp.float32(xv.shape[1])
    w = s2 - (2.0 * s1) * p + n * (p * p)
    rinv_ref[...] = 0.6931471805599453 / w


def _argmax_first(t, idx_ref):
    m = jnp.max(t, axis=1, keepdims=True)
    cols = lax.broadcasted_iota(jnp.int32, t.shape, 1)
    idx_ref[...] = jnp.min(jnp.where(t == m, cols, jnp.int32(2 ** 30)),
                           axis=1, keepdims=True)


def _bits_to_t(bits, rinv2):
    fb = lax.bitcast_convert_type(
        (bits >> jnp.uint32(9)) | jnp.uint32(0x3F800000), jnp.float32)
    return jnp.log2(fb - 1.0) * rinv2


def _sample_body(rinv2_ref, idx_ref):
    i0 = pl.program_id(0) * R
    base = (i0 * P + _KS1).astype(jnp.uint32)
    n = (lax.broadcasted_iota(jnp.uint32, (R, P), 0) * jnp.uint32(P)
         + lax.broadcasted_iota(jnp.uint32, (R, P), 1)) + base
    t = _bits_to_t(_threefry_bits_pre(n), rinv2_ref[...])
    _argmax_first(t, idx_ref)


def _sample_scbits_body(rinv2_ref, bits_ref, idx_ref):
    t = _bits_to_t(bits_ref[...], rinv2_ref[...])
    _argmax_first(t, idx_ref)


RTC = 5248
RS = P - RTC
RPT = RS // NW
RB2 = 64

_CHUNKS = BPW // 128


@functools.cache
def _make_sc_bits():
    mesh = plsc.VectorSubcoreMesh(core_axis_name="c", subcore_axis_name="s")

    @functools.partial(
        pl.kernel,
        mesh=mesh,
        out_type=jax.ShapeDtypeStruct((RS, P), jnp.uint32),
        scratch_types=[pltpu.VMEM((2, P), jnp.uint32), pltpu.SemaphoreType.DMA],
    )
    def _sc_bits(out_hbm, bufs_v, sem):
        wid = lax.axis_index("s") * 2 + lax.axis_index("c")
        lane = lax.iota(jnp.uint32, 16)

        def row_body(r, carry):
            lrow = wid * RPT + r
            slot = lax.rem(r, 2)
            buf = bufs_v.at[slot]
            @pl.when(r >= 2)
            def _():
                pltpu.make_async_copy(buf, out_hbm.at[lrow], sem).wait()

            nbase = ((RTC + lrow) * P + _KS1).astype(jnp.uint32)

            def grp(g, carry2):
                goff = (g * 256).astype(jnp.uint32)
                for k in range(16):
                    x1 = lane + (nbase + goff + jnp.uint32(k * 16))
                    buf[pl.ds(g * 256 + k * 16, 16)] = _threefry_bits_pre(x1)
                return carry2

            lax.fori_loop(0, P // 256, grp, 0)
            pltpu.async_copy(buf, out_hbm.at[lrow], sem)
            return carry

        lax.fori_loop(0, RPT, row_body, 0)
        pltpu.make_async_copy(bufs_v.at[0], out_hbm.at[wid * RPT], sem).wait()
        pltpu.make_async_copy(bufs_v.at[1], out_hbm.at[wid * RPT], sem).wait()

    return _sc_bits


@functools.cache
def _make_gather_mean():
    mesh = plsc.VectorSubcoreMesh(core_axis_name="c", subcore_axis_name="s")

    @functools.partial(
        pl.kernel,
        mesh=mesh,
        out_type=jax.ShapeDtypeStruct((NW, 16), jnp.float32),
        scratch_types=[
            pltpu.VMEM((_CHUNKS, 128), jnp.int32),
            pltpu.VMEM((_CHUNKS, 128, 128), jnp.float32),
            pltpu.VMEM((16,), jnp.float32),
            pltpu.SemaphoreType.DMA,
        ],
    )
    def _gather_mean(table_hbm, idx_hbm, out_hbm, idx_v, rows_v, acc_v, sem):
        wid = lax.axis_index("s") * 2 + lax.axis_index("c")
        pltpu.sync_copy(idx_hbm.at[pl.ds(wid * _CHUNKS, _CHUNKS)], idx_v)
        for j in range(_CHUNKS):
            pltpu.async_copy(table_hbm.at[idx_v.at[j]], rows_v.at[j], sem).wait()
        acc = jnp.zeros((16,), jnp.float32)
        for j in range(_CHUNKS):
            for g in range(128):
                acc = acc + rows_v[j, g, pl.ds(0, 16)]
        acc_v[...] = acc
        pltpu.sync_copy(acc_v, out_hbm.at[wid])

    return _gather_mean


def kernel(inputs, state_vector, transition_matrix, process_noise_cov, forward_matrix):
    x = inputs.astype(jnp.float32)
    noise = jax.random.normal(jax.random.key(1), state_vector.shape, jnp.float32)
    svt = state_vector.T
    nzt = noise.T

    updt, rinv = pl.pallas_call(
        _prologue_body,
        in_specs=[
            pl.BlockSpec(memory_space=pltpu.VMEM),
            pl.BlockSpec(memory_space=pltpu.VMEM),
            pl.BlockSpec(memory_space=pltpu.VMEM),
            pl.BlockSpec(memory_space=pltpu.SMEM),
            pl.BlockSpec(memory_space=pltpu.SMEM),
            pl.BlockSpec(memory_space=pltpu.SMEM),
        ],
        out_specs=[
            pl.BlockSpec(memory_space=pltpu.VMEM),
            pl.BlockSpec(memory_space=pltpu.VMEM),
        ],
        out_shape=[
            jax.ShapeDtypeStruct((3, P), jnp.float32),
            jax.ShapeDtypeStruct((1, P), jnp.float32),
        ],
    )(x, svt, nzt, transition_matrix, process_noise_cov, forward_matrix)

    scbits = _make_sc_bits()()

    idx1 = pl.pallas_call(
        _sample_body,
        grid=(RTC // R,),
        in_specs=[pl.BlockSpec((1, P), lambda s: (0, 0))],
        out_specs=pl.BlockSpec((R, 1), lambda s: (s, 0)),
        out_shape=jax.ShapeDtypeStruct((RTC, 1), jnp.int32),
    )(rinv)

    idx2 = pl.pallas_call(
        _sample_scbits_body,
        grid=(RS // RB2,),
        in_specs=[pl.BlockSpec((1, P), lambda s: (0, 0)),
                  pl.BlockSpec((RB2, P), lambda s: (s, 0))],
        out_specs=pl.BlockSpec((RB2, 1), lambda s: (s, 0)),
        out_shape=jax.ShapeDtypeStruct((RS, 1), jnp.int32),
    )(rinv, scbits)

    idx2d = jnp.concatenate([idx1, idx2], axis=0)
    table = jnp.zeros((P, 128), jnp.float32).at[:, :3].set(updt.T)
    partials = _make_gather_mean()(table, idx2d.reshape(NW * _CHUNKS, 128))
    return jnp.sum(partials, axis=0)[:3] / jnp.float32(P)
